# baseline (speedup 1.0000x reference)
import functools

import jax
import jax.numpy as jnp
from jax.experimental import pallas as pl
from jax.experimental.pallas import tpu as pltpu

_F32 = jnp.float32

_EF4 = ((1.0, 0.0), (-1.0, 0.0), (0.0, 1.0), (0.0, -1.0))


def _dot(a, b):
    return jnp.dot(a.astype(jnp.bfloat16), b.astype(jnp.bfloat16),
                   preferred_element_type=_F32)


def _gru_core(nx, h, hid, w4, nnb, rdeg, gwih, gbih, gwhh, gbhh):
    n = hid.shape[0]
    row = jax.lax.broadcasted_iota(jnp.int32, (n, 1), 0)
    j = row % nx
    i = (row // nx) % nx
    m0 = (j >= 1)
    m1 = (j <= nx - 2)
    m2 = (i >= 1)
    m3 = (i <= nx - 2)
    z1 = jnp.zeros((1, h), _F32)
    znx = jnp.zeros((nx, h), _F32)
    s0 = jnp.where(m0, jnp.concatenate([z1, hid[:-1]], axis=0), 0.0)
    s1 = jnp.where(m1, jnp.concatenate([hid[1:], z1], axis=0), 0.0)
    s2 = jnp.where(m2, jnp.concatenate([znx, hid[:-nx]], axis=0), 0.0)
    s3 = jnp.where(m3, jnp.concatenate([hid[nx:], znx], axis=0), 0.0)
    xcat = jnp.concatenate([s0, s1, s2, s3], axis=1)
    ssum = _dot(xcat, w4)
    m = jnp.maximum(ssum * rdeg + nnb, 0.0)
    gi = _dot(m, gwih) + gbih
    gh = _dot(hid, gwhh) + gbhh
    r = jax.nn.sigmoid(gi[:, :h] + gh[:, :h])
    z = jax.nn.sigmoid(gi[:, h:2 * h] + gh[:, h:2 * h])
    nn = jnp.tanh(gi[:, 2 * h:] + r * gh[:, 2 * h:])
    return (1.0 - z) * nn + z * hid


def _unet_kern(t, nx, h1, h2,
               x_ref,
               aw1, ab1, aw2, ab2, a4, anb, awih, abih, awhh, abhh,
               bw1, bb1, bw2, bb2, b4, bnb, bwih, bbih, bwhh, bbhh,
               cwa, cwb, cb1, cw2, cb2, c4, cnb, cwih, cbih, cwhh, cbhh,
               rdf_ref, rdc_ref,
               o_ref, spool_ref, sup_ref):
    nf = t * nx * nx
    nh = nx // 2
    nc = t * nh * nh

    l1 = jnp.maximum(_dot(x_ref[...], aw1[...]) + ab1[...], 0.0)
    hid = _dot(l1, aw2[...]) + ab2[...]
    bp = _gru_core(nx, h1, hid, a4[...], anb[...], rdf_ref[...], awih[...],
                   abih[...], awhh[...], abhh[...])

    z1 = jnp.zeros((1, h1), _F32)
    spool_ref[...] = bp + jnp.concatenate([bp[1:], z1], axis=0)
    t1 = spool_ref[pl.Slice(0, nf // 2, 2), :]
    t4 = t1.reshape(t * nx // 2, 2, nh, h1)
    d = ((t4[:, 0] + t4[:, 1]) * 0.25).reshape(nc, h1)

    l1b = jnp.maximum(_dot(d, bw1[...]) + bb1[...], 0.0)
    hidb = _dot(l1b, bw2[...]) + bb2[...]
    h2v = _gru_core(nh, h2, hidb, b4[...], bnb[...], rdc_ref[...],
                    bwih[...], bbih[...], bwhh[...], bbhh[...])

    u3 = h2v.reshape(t * nh, 1, nh, h2)
    ui = jnp.concatenate([u3, u3], axis=1).reshape(nf // 2, h2)
    v = _dot(ui, cwb[...])
    sup_ref[pl.Slice(0, nf // 2, 2), :] = v
    sup_ref[pl.Slice(1, nf // 2, 2), :] = v

    pre = _dot(bp, cwa[...]) + sup_ref[...] + cb1[...]
    l1c = jnp.maximum(pre, 0.0)
    hidc = _dot(l1c, cw2[...]) + cb2[...]
    o_ref[...] = _gru_core(nx, h1, hidc, c4[...], cnb[...], rdf_ref[...],
                           cwih[...], cbih[...], cwhh[...], cbhh[...])


def _edge_w4(p, h):
    ef = jnp.asarray(_EF4, _F32)
    a = jnp.maximum(ef @ p['eW1'] + p['eb1'], 0.0)
    w = (a @ p['eW2'] + p['eb2']).reshape(4, h, h)
    return w.reshape(4 * h, h)


def _row(v):
    return v.reshape(1, -1)


def _rdeg(t, nx):
    idx = jnp.arange(t * nx * nx, dtype=jnp.int32)
    j = idx % nx
    i = (idx // nx) % nx
    deg = ((j >= 1).astype(_F32) + (j <= nx - 2).astype(_F32)
           + (i >= 1).astype(_F32) + (i <= nx - 2).astype(_F32))
    return (1.0 / deg).reshape(-1, 1)


def _block_args(p, h):
    return (p['pW1'], _row(p['pb1']), p['pW2'], _row(p['pb2']),
            _edge_w4(p, h), _row(p['nnb']), p['gWih'], _row(p['gbih']),
            p['gWhh'], _row(p['gbhh']))


def kernel(inputs, params):
    b, t, nx, ny, c = inputs.shape
    h1 = params['c1']['pb2'].shape[0]
    h2 = params['lw']['pb2'].shape[0]
    p2 = params['c2']
    cwa = p2['pW1'][:h1]
    cwb = params['upW'] @ p2['pW1'][h1:]
    cb1 = p2['pb1'] + params['upb'] @ p2['pW1'][h1:]
    c2_args = (cwa, cwb, _row(cb1), p2['pW2'], _row(p2['pb2']),
               _edge_w4(p2, h1), _row(p2['nnb']), p2['gWih'],
               _row(p2['gbih']), p2['gWhh'], _row(p2['gbhh']))
    fn = functools.partial(_unet_kern, t, nx, h1, h2)
    nf = t * nx * ny
    call = pl.pallas_call(
        fn,
        out_shape=jax.ShapeDtypeStruct((nf, h1), _F32),
        scratch_shapes=[pltpu.VMEM((nf, h1), _F32),
                        pltpu.VMEM((nf, h1), _F32)],
    )
    outs = []
    for bi in range(b):
        x = inputs[bi].reshape(t * nx * ny, c)
        h3 = call(x, *_block_args(params['c1'], h1),
                  *_block_args(params['lw'], h2), *c2_args,
                  _rdeg(t, nx), _rdeg(t, nx // 2))
        outs.append(h3.reshape(t, nx, ny, h1))
    return jnp.stack(outs, 0)


# speedup vs baseline: 1.0072x; 1.0072x over previous
import functools

import jax
import jax.numpy as jnp
from jax.experimental import pallas as pl
from jax.experimental.pallas import tpu as pltpu

_F32 = jnp.float32

_EF4 = ((1.0, 0.0), (-1.0, 0.0), (0.0, 1.0), (0.0, -1.0))


def _dot(a, b):
    return jnp.dot(a, b, preferred_element_type=_F32)


def _gru_core(nx, h, hid, w4, nnb, rdeg, gwih, gbih, gwhh, gbhh):
    n = hid.shape[0]
    row = jax.lax.broadcasted_iota(jnp.int32, (n, 1), 0)
    j = row % nx
    i = (row // nx) % nx
    m0 = (j >= 1)
    m1 = (j <= nx - 2)
    m2 = (i >= 1)
    m3 = (i <= nx - 2)
    z1 = jnp.zeros((1, h), _F32)
    znx = jnp.zeros((nx, h), _F32)
    s0 = jnp.where(m0, jnp.concatenate([z1, hid[:-1]], axis=0), 0.0)
    s1 = jnp.where(m1, jnp.concatenate([hid[1:], z1], axis=0), 0.0)
    s2 = jnp.where(m2, jnp.concatenate([znx, hid[:-nx]], axis=0), 0.0)
    s3 = jnp.where(m3, jnp.concatenate([hid[nx:], znx], axis=0), 0.0)
    xcat = jnp.concatenate([s0, s1, s2, s3], axis=1)
    ssum = _dot(xcat, w4)
    m = jnp.maximum(ssum * rdeg + nnb, 0.0)
    gi = _dot(m, gwih) + gbih
    gh = _dot(hid, gwhh) + gbhh
    r = jax.nn.sigmoid(gi[:, :h] + gh[:, :h])
    z = jax.nn.sigmoid(gi[:, h:2 * h] + gh[:, h:2 * h])
    nn = jnp.tanh(gi[:, 2 * h:] + r * gh[:, 2 * h:])
    return (1.0 - z) * nn + z * hid


def _unet_kern(t, nx, h1, h2,
               x_ref,
               aw1, ab1, aw2, ab2, a4, anb, awih, abih, awhh, abhh,
               bw1, bb1, bw2, bb2, b4, bnb, bwih, bbih, bwhh, bbhh,
               cwa, cwb, cb1, cw2, cb2, c4, cnb, cwih, cbih, cwhh, cbhh,
               rdf_ref, rdc_ref,
               o_ref, spool_ref, sup_ref):
    nf = t * nx * nx
    nh = nx // 2
    nc = t * nh * nh

    l1 = jnp.maximum(_dot(x_ref[...], aw1[...]) + ab1[...], 0.0)
    hid = _dot(l1, aw2[...]) + ab2[...]
    bp = _gru_core(nx, h1, hid, a4[...], anb[...], rdf_ref[...], awih[...],
                   abih[...], awhh[...], abhh[...])

    z1 = jnp.zeros((1, h1), _F32)
    spool_ref[...] = bp + jnp.concatenate([bp[1:], z1], axis=0)
    t1 = spool_ref[pl.Slice(0, nf // 2, 2), :]
    t4 = t1.reshape(t * nx // 2, 2, nh, h1)
    d = ((t4[:, 0] + t4[:, 1]) * 0.25).reshape(nc, h1)

    l1b = jnp.maximum(_dot(d, bw1[...]) + bb1[...], 0.0)
    hidb = _dot(l1b, bw2[...]) + bb2[...]
    h2v = _gru_core(nh, h2, hidb, b4[...], bnb[...], rdc_ref[...],
                    bwih[...], bbih[...], bwhh[...], bbhh[...])

    u3 = h2v.reshape(t * nh, 1, nh, h2)
    ui = jnp.concatenate([u3, u3], axis=1).reshape(nf // 2, h2)
    v = _dot(ui, cwb[...])
    sup_ref[pl.Slice(0, nf // 2, 2), :] = v
    sup_ref[pl.Slice(1, nf // 2, 2), :] = v

    pre = _dot(bp, cwa[...]) + sup_ref[...] + cb1[...]
    l1c = jnp.maximum(pre, 0.0)
    hidc = _dot(l1c, cw2[...]) + cb2[...]
    o_ref[...] = _gru_core(nx, h1, hidc, c4[...], cnb[...], rdf_ref[...],
                           cwih[...], cbih[...], cwhh[...], cbhh[...])


def _edge_w4(p, h):
    ef = jnp.asarray(_EF4, _F32)
    a = jnp.maximum(ef @ p['eW1'] + p['eb1'], 0.0)
    w = (a @ p['eW2'] + p['eb2']).reshape(4, h, h)
    return w.reshape(4 * h, h)


def _row(v):
    return v.reshape(1, -1)


def _rdeg(t, nx):
    idx = jnp.arange(t * nx * nx, dtype=jnp.int32)
    j = idx % nx
    i = (idx // nx) % nx
    deg = ((j >= 1).astype(_F32) + (j <= nx - 2).astype(_F32)
           + (i >= 1).astype(_F32) + (i <= nx - 2).astype(_F32))
    return (1.0 / deg).reshape(-1, 1)


def _block_args(p, h):
    return (p['pW1'], _row(p['pb1']), p['pW2'], _row(p['pb2']),
            _edge_w4(p, h), _row(p['nnb']), p['gWih'], _row(p['gbih']),
            p['gWhh'], _row(p['gbhh']))


def kernel(inputs, params):
    b, t, nx, ny, c = inputs.shape
    h1 = params['c1']['pb2'].shape[0]
    h2 = params['lw']['pb2'].shape[0]
    p2 = params['c2']
    cwa = p2['pW1'][:h1]
    cwb = params['upW'] @ p2['pW1'][h1:]
    cb1 = p2['pb1'] + params['upb'] @ p2['pW1'][h1:]
    c2_args = (cwa, cwb, _row(cb1), p2['pW2'], _row(p2['pb2']),
               _edge_w4(p2, h1), _row(p2['nnb']), p2['gWih'],
               _row(p2['gbih']), p2['gWhh'], _row(p2['gbhh']))
    fn = functools.partial(_unet_kern, t, nx, h1, h2)
    nf = t * nx * ny
    call = pl.pallas_call(
        fn,
        out_shape=jax.ShapeDtypeStruct((nf, h1), _F32),
        scratch_shapes=[pltpu.VMEM((nf, h1), _F32),
                        pltpu.VMEM((nf, h1), _F32)],
    )
    outs = []
    for bi in range(b):
        x = inputs[bi].reshape(t * nx * ny, c)
        h3 = call(x, *_block_args(params['c1'], h1),
                  *_block_args(params['lw'], h2), *c2_args,
                  _rdeg(t, nx), _rdeg(t, nx // 2))
        outs.append(h3.reshape(t, nx, ny, h1))
    return jnp.stack(outs, 0)


# speedup vs baseline: 1.2504x; 1.2415x over previous
import functools

import jax
import jax.numpy as jnp
from jax.experimental import pallas as pl
from jax.experimental.pallas import tpu as pltpu

_F32 = jnp.float32

_EF4 = ((1.0, 0.0), (-1.0, 0.0), (0.0, 1.0), (0.0, -1.0))


def _dot(a, b):
    return jnp.dot(a, b, preferred_element_type=_F32)


def _gru_core(nx, h, hid, w4, nnb, gwih, gbih, gwhh, gbhh):
    n = hid.shape[0]
    row = jax.lax.broadcasted_iota(jnp.int32, (n, 1), 0)
    j = row % nx
    i = (row // nx) % nx
    m0 = (j >= 1)
    m1 = (j <= nx - 2)
    m2 = (i >= 1)
    m3 = (i <= nx - 2)
    z1 = jnp.zeros((1, h), _F32)
    znx = jnp.zeros((nx, h), _F32)
    s0 = jnp.where(m0, jnp.concatenate([z1, hid[:-1]], axis=0), 0.0)
    s1 = jnp.where(m1, jnp.concatenate([hid[1:], z1], axis=0), 0.0)
    s2 = jnp.where(m2, jnp.concatenate([znx, hid[:-nx]], axis=0), 0.0)
    s3 = jnp.where(m3, jnp.concatenate([hid[nx:], znx], axis=0), 0.0)
    xcat = jnp.concatenate([s0, s1, s2, s3], axis=1)
    ssum = _dot(xcat, w4)
    deg = (m0.astype(_F32) + m1.astype(_F32) + m2.astype(_F32)
           + m3.astype(_F32))
    m = jnp.maximum(ssum * (1.0 / deg) + nnb, 0.0)
    gi = _dot(m, gwih) + gbih
    gh = _dot(hid, gwhh) + gbhh
    r = jax.nn.sigmoid(gi[:, :h] + gh[:, :h])
    z = jax.nn.sigmoid(gi[:, h:2 * h] + gh[:, h:2 * h])
    nn = jnp.tanh(gi[:, 2 * h:] + r * gh[:, 2 * h:])
    return (1.0 - z) * nn + z * hid


def _unet_kern(t, nx, h1, h2,
               x_ref,
               aw1, ab1, aw2, ab2, a4, anb, awih, abih, awhh, abhh,
               bw1, bb1, bw2, bb2, b4, bnb, bwih, bbih, bwhh, bbhh,
               cwa, cwb, cb1, cw2, cb2, c4, cnb, cwih, cbih, cwhh, cbhh,
               o_ref, spool_ref, sup_ref):
    nf = t * nx * nx
    nh = nx // 2
    nc = t * nh * nh

    l1 = jnp.maximum(_dot(x_ref[...], aw1[...]) + ab1[...], 0.0)
    hid = _dot(l1, aw2[...]) + ab2[...]
    bp = _gru_core(nx, h1, hid, a4[...], anb[...], awih[...],
                   abih[...], awhh[...], abhh[...])

    z1 = jnp.zeros((1, h1), _F32)
    spool_ref[...] = bp + jnp.concatenate([bp[1:], z1], axis=0)
    t1 = spool_ref[pl.Slice(0, nf // 2, 2), :]
    t4 = t1.reshape(t * nx // 2, 2, nh, h1)
    d = ((t4[:, 0] + t4[:, 1]) * 0.25).reshape(nc, h1)

    l1b = jnp.maximum(_dot(d, bw1[...]) + bb1[...], 0.0)
    hidb = _dot(l1b, bw2[...]) + bb2[...]
    h2v = _gru_core(nh, h2, hidb, b4[...], bnb[...],
                    bwih[...], bbih[...], bwhh[...], bbhh[...])

    u3 = h2v.reshape(t * nh, 1, nh, h2)
    ui = jnp.concatenate([u3, u3], axis=1).reshape(nf // 2, h2)
    v = _dot(ui, cwb[...])
    sup_ref[pl.Slice(0, nf // 2, 2), :] = v
    sup_ref[pl.Slice(1, nf // 2, 2), :] = v

    pre = _dot(bp, cwa[...]) + sup_ref[...] + cb1[...]
    l1c = jnp.maximum(pre, 0.0)
    hidc = _dot(l1c, cw2[...]) + cb2[...]
    o_ref[...] = _gru_core(nx, h1, hidc, c4[...], cnb[...],
                           cwih[...], cbih[...], cwhh[...], cbhh[...])


def _edge_w4(p, h):
    ef = jnp.asarray(_EF4, _F32)
    a = jnp.maximum(ef @ p['eW1'] + p['eb1'], 0.0)
    w = (a @ p['eW2'] + p['eb2']).reshape(4, h, h)
    return w.reshape(4 * h, h)


def _row(v):
    return v.reshape(1, -1)


def _block_args(p, h):
    return (p['pW1'], _row(p['pb1']), p['pW2'], _row(p['pb2']),
            _edge_w4(p, h), _row(p['nnb']), p['gWih'], _row(p['gbih']),
            p['gWhh'], _row(p['gbhh']))


def kernel(inputs, params):
    b, t, nx, ny, c = inputs.shape
    h1 = params['c1']['pb2'].shape[0]
    h2 = params['lw']['pb2'].shape[0]
    p2 = params['c2']
    cwa = p2['pW1'][:h1]
    cwb = params['upW'] @ p2['pW1'][h1:]
    cb1 = p2['pb1'] + params['upb'] @ p2['pW1'][h1:]
    c2_args = (cwa, cwb, _row(cb1), p2['pW2'], _row(p2['pb2']),
               _edge_w4(p2, h1), _row(p2['nnb']), p2['gWih'],
               _row(p2['gbih']), p2['gWhh'], _row(p2['gbhh']))
    fn = functools.partial(_unet_kern, t, nx, h1, h2)
    nf = t * nx * ny
    call = pl.pallas_call(
        fn,
        out_shape=jax.ShapeDtypeStruct((nf, h1), _F32),
        scratch_shapes=[pltpu.VMEM((nf, h1), _F32),
                        pltpu.VMEM((nf, h1), _F32)],
    )
    outs = []
    for bi in range(b):
        x = inputs[bi].reshape(t * nx * ny, c)
        h3 = call(x, *_block_args(params['c1'], h1),
                  *_block_args(params['lw'], h2), *c2_args)
        outs.append(h3.reshape(t, nx, ny, h1))
    return jnp.stack(outs, 0)


# speedup vs baseline: 1.2605x; 1.0081x over previous
import functools

import jax
import jax.numpy as jnp
from jax.experimental import pallas as pl
from jax.experimental.pallas import tpu as pltpu

_F32 = jnp.float32

_EF4 = ((1.0, 0.0), (-1.0, 0.0), (0.0, 1.0), (0.0, -1.0))


def _dot(a, b):
    return jnp.dot(a, b, preferred_element_type=_F32)


def _gru_core(nx, h, hid, w4, nnb, gwih, gbih, gwhh, gbhh):
    n = hid.shape[0]
    row = jax.lax.broadcasted_iota(jnp.int32, (n, 1), 0)
    j = row % nx
    i = (row // nx) % nx
    m0 = (j >= 1)
    m1 = (j <= nx - 2)
    m2 = (i >= 1)
    m3 = (i <= nx - 2)
    z1 = jnp.zeros((1, h), _F32)
    znx = jnp.zeros((nx, h), _F32)
    s0 = jnp.where(m0, jnp.concatenate([z1, hid[:-1]], axis=0), 0.0)
    s1 = jnp.where(m1, jnp.concatenate([hid[1:], z1], axis=0), 0.0)
    s2 = jnp.where(m2, jnp.concatenate([znx, hid[:-nx]], axis=0), 0.0)
    s3 = jnp.where(m3, jnp.concatenate([hid[nx:], znx], axis=0), 0.0)
    xcat = jnp.concatenate([s0, s1, s2, s3], axis=1)
    ssum = _dot(xcat, w4)
    deg = (m0.astype(_F32) + m1.astype(_F32) + m2.astype(_F32)
           + m3.astype(_F32))
    m = jnp.maximum(ssum * (1.0 / deg) + nnb, 0.0)
    gi = _dot(m, gwih) + gbih
    gh = _dot(hid, gwhh) + gbhh
    r = jax.nn.sigmoid(gi[:, :h] + gh[:, :h])
    z = jax.nn.sigmoid(gi[:, h:2 * h] + gh[:, h:2 * h])
    nn = jnp.tanh(gi[:, 2 * h:] + r * gh[:, 2 * h:])
    return (1.0 - z) * nn + z * hid


def _unet_kern(t, nx, h1, h2,
               x_ref,
               aw1, ab1, aw2, ab2, a4, anb, awih, abih, awhh, abhh,
               bw1, bb1, bw2, bb2, b4, bnb, bwih, bbih, bwhh, bbhh,
               cwa, cwb, cb1, cw2, cb2, c4, cnb, cwih, cbih, cwhh, cbhh,
               o_ref, spool_ref, sup_ref):
    nf = t * nx * nx
    nh = nx // 2
    nc = t * nh * nh

    l1 = jnp.maximum(_dot(x_ref[...], aw1[...]) + ab1[...], 0.0)
    hid = _dot(l1, aw2[...]) + ab2[...]
    bp = _gru_core(nx, h1, hid, a4[...], anb[...], awih[...],
                   abih[...], awhh[...], abhh[...])

    z1 = jnp.zeros((1, h1), _F32)
    spool_ref[...] = bp + jnp.concatenate([bp[1:], z1], axis=0)
    t1 = spool_ref[pl.Slice(0, nf // 2, 2), :]
    t4 = t1.reshape(t * nx // 2, 2, nh, h1)
    d = ((t4[:, 0] + t4[:, 1]) * 0.25).reshape(nc, h1)

    l1b = jnp.maximum(_dot(d, bw1[...]) + bb1[...], 0.0)
    hidb = _dot(l1b, bw2[...]) + bb2[...]
    h2v = _gru_core(nh, h2, hidb, b4[...], bnb[...],
                    bwih[...], bbih[...], bwhh[...], bbhh[...])

    u3 = h2v.reshape(t * nh, 1, nh, h2)
    ui = jnp.concatenate([u3, u3], axis=1).reshape(nf // 2, h2)
    v = _dot(ui, cwb[...])
    sup_ref[pl.Slice(0, nf // 2, 2), :] = v
    sup_ref[pl.Slice(1, nf // 2, 2), :] = v

    pre = _dot(bp, cwa[...]) + sup_ref[...] + cb1[...]
    l1c = jnp.maximum(pre, 0.0)
    hidc = _dot(l1c, cw2[...]) + cb2[...]
    o_ref[...] = _gru_core(nx, h1, hidc, c4[...], cnb[...],
                           cwih[...], cbih[...], cwhh[...], cbhh[...])


def _edge_w4(p, h):
    ef = jnp.asarray(_EF4, _F32)
    a = jnp.maximum(ef @ p['eW1'] + p['eb1'], 0.0)
    w = (a @ p['eW2'] + p['eb2']).reshape(4, h, h)
    return w.reshape(4 * h, h)


def _row(v):
    return v.reshape(1, -1)


def _block_args(p, h):
    return (p['pW1'], _row(p['pb1']), p['pW2'], _row(p['pb2']),
            _edge_w4(p, h), _row(p['nnb']), p['gWih'], _row(p['gbih']),
            p['gWhh'], _row(p['gbhh']))


def kernel(inputs, params):
    b, t, nx, ny, c = inputs.shape
    h1 = params['c1']['pb2'].shape[0]
    h2 = params['lw']['pb2'].shape[0]
    p2 = params['c2']
    cwa = p2['pW1'][:h1]
    cwb = params['upW'] @ p2['pW1'][h1:]
    cb1 = p2['pb1'] + params['upb'] @ p2['pW1'][h1:]
    c2_args = (cwa, cwb, _row(cb1), p2['pW2'], _row(p2['pb2']),
               _edge_w4(p2, h1), _row(p2['nnb']), p2['gWih'],
               _row(p2['gbih']), p2['gWhh'], _row(p2['gbhh']))
    gsplit = 2
    th = t // gsplit
    fn = functools.partial(_unet_kern, th, nx, h1, h2)
    nf = t * nx * ny
    nfh = nf // gsplit
    def _wspec(a):
        return pl.BlockSpec(a.shape, lambda g: (0,) * a.ndim)
    call = lambda xx, *ws: pl.pallas_call(
        fn,
        grid=(gsplit,),
        in_specs=[pl.BlockSpec((nfh, c), lambda g: (g, 0))]
        + [_wspec(w) for w in ws],
        out_specs=pl.BlockSpec((nfh, h1), lambda g: (g, 0)),
        out_shape=jax.ShapeDtypeStruct((nf, h1), _F32),
        scratch_shapes=[pltpu.VMEM((nfh, h1), _F32),
                        pltpu.VMEM((nfh, h1), _F32)],
        compiler_params=pltpu.CompilerParams(
            dimension_semantics=("parallel",)),
    )(xx, *ws)
    outs = []
    for bi in range(b):
        x = inputs[bi].reshape(t * nx * ny, c)
        h3 = call(x, *_block_args(params['c1'], h1),
                  *_block_args(params['lw'], h2), *c2_args)
        outs.append(h3.reshape(t, nx, ny, h1))
    return jnp.stack(outs, 0)
